# baseline (speedup 1.0000x reference)
import functools

import jax
import jax.numpy as jnp
from jax import lax
from jax.experimental import pallas as pl
from jax.experimental.pallas import tpu as pltpu
from jax.experimental.pallas import tpu_sc as plsc

_BATCH = 1024
_NUM_CORES = 1
_NUM_SUBCORES = 16
_NW = _NUM_CORES * _NUM_SUBCORES
_CHUNK = _BATCH // _NW
_LANES = 16

_mesh = plsc.VectorSubcoreMesh(
    core_axis_name="c", subcore_axis_name="s", num_cores=_NUM_CORES)


@functools.partial(
    pl.kernel,
    mesh=_mesh,
    out_type=jax.ShapeDtypeStruct((_BATCH,), jnp.int32),
    scratch_types=[pltpu.VMEM((_BATCH,), jnp.int32)],
)
def _fix_labels(labels_hbm, out_hbm, buf):
    wid = lax.axis_index("s") * _NUM_CORES + lax.axis_index("c")

    @pl.when(wid == 0)
    def _():
        pltpu.sync_copy(labels_hbm, buf)
        neg_one = jnp.full((_LANES,), -1, jnp.int32)
        for i in range(_BATCH // _LANES):
            v = buf[pl.ds(i * _LANES, _LANES)]
            buf[pl.ds(i * _LANES, _LANES)] = jnp.where(v >= 0, v, neg_one)
        pltpu.sync_copy(buf, out_hbm)


def kernel(features, labels):
    del features
    return _fix_labels(labels.reshape(-1))


# speedup vs baseline: 1.0340x; 1.0340x over previous
import functools

import jax
import jax.numpy as jnp
from jax import lax
from jax.experimental import pallas as pl
from jax.experimental.pallas import tpu as pltpu
from jax.experimental.pallas import tpu_sc as plsc

_BATCH = 1024
_NUM_CORES = 1
_NUM_SUBCORES = 1
_LANES = 16

_mesh = plsc.VectorSubcoreMesh(
    core_axis_name="c", subcore_axis_name="s",
    num_cores=_NUM_CORES, num_subcores=_NUM_SUBCORES)


@functools.partial(
    pl.kernel,
    mesh=_mesh,
    out_type=jax.ShapeDtypeStruct((_BATCH,), jnp.int32),
    scratch_types=[pltpu.VMEM((_BATCH,), jnp.int32)],
)
def _fix_labels(labels_hbm, out_hbm, buf):
    pltpu.sync_copy(labels_hbm, buf)
    neg_one = jnp.full((_LANES,), -1, jnp.int32)
    for i in range(_BATCH // _LANES):
        v = buf[pl.ds(i * _LANES, _LANES)]
        buf[pl.ds(i * _LANES, _LANES)] = jnp.where(v >= 0, v, neg_one)
    pltpu.sync_copy(buf, out_hbm)


def kernel(features, labels):
    del features
    return _fix_labels(labels.reshape(-1))
